# baseline (speedup 1.0000x reference)
import functools

import jax
import jax.numpy as jnp
from jax import lax
from jax.experimental import pallas as pl
from jax.experimental.pallas import tpu as pltpu
from jax.experimental.pallas import tpu_sc as plsc

NC = 2
NS = 16
NW = NC * NS
CHUNK = 80


ROWBLK = 80
GRP = 1


def _seg_sum_body(n_nodes, feat, edges_per_w, h_hbm, src_hbm, dst_hbm,
                  out_hbm, agg_sh, flat_v, didx_v, rows_v,
                  gsem_a, gsem_b, gsem_c, ssem_a, ssem_b, ssem_c):
    c = lax.axis_index("c")
    s = lax.axis_index("s")
    wid = c * NS + s
    nchunks = edges_per_w // CHUNK
    ngroups = nchunks // GRP
    base = wid * edges_per_w

    nblk = n_nodes // ROWBLK
    blk_iters = (nblk + NS - 1) // NS

    zsrc = rows_v.at[0]

    def zero_row(r, _):
        for j in range(feat // 16):
            zsrc[r, pl.ds(j * 16, 16)] = jnp.zeros((16,), jnp.float32)
        return 0
    lax.fori_loop(0, ROWBLK, zero_row, 0)

    def zero_blk(k, _):
        b = s + k * NS

        @pl.when(b < nblk)
        def _():
            pltpu.sync_copy(zsrc, agg_sh.at[pl.ds(b * ROWBLK, ROWBLK)])
        return 0
    lax.fori_loop(0, blk_iters, zero_blk, 0)

    pltpu.sync_copy(dst_hbm.at[pl.ds(base, edges_per_w)], didx_v)
    pltpu.sync_copy(src_hbm.at[pl.ds(base, edges_per_w)], flat_v)

    plsc.subcore_barrier()

    gsems = (gsem_a, gsem_b, gsem_c)
    ssems = (ssem_a, ssem_b, ssem_c)

    def fire_gather(k, m):
        pltpu.async_copy(
            h_hbm.at[flat_v.at[pl.ds(k * CHUNK, CHUNK)]], rows_v.at[m], gsems[m])

    def drain_gather(k, m):
        pltpu.make_async_copy(
            h_hbm.at[flat_v.at[pl.ds(k * CHUNK, CHUNK)]], rows_v.at[m],
            gsems[m]).wait()

    def fire_scatter(k, m):
        pltpu.async_copy(rows_v.at[m],
                         agg_sh.at[didx_v.at[pl.ds(k * CHUNK, CHUNK)]],
                         ssems[m], add=True)

    def drain_scatter(k, m):
        pltpu.make_async_copy(rows_v.at[m],
                              agg_sh.at[didx_v.at[pl.ds(k * CHUNK, CHUNK)]],
                              ssems[m]).wait()

    fire_gather(0, 0)

    def pipe(i, _):
        for m in range(3):
            k = 3 * i + m
            mn = (m + 1) % 3

            @pl.when(k >= 2)
            def _():
                drain_scatter(k - 2, mn)
            fire_gather(k + 1, mn)
            drain_gather(k, m)
            fire_scatter(k, m)
        return 0
    nfull = (nchunks - 1) // 3
    lax.fori_loop(0, nfull, pipe, 0)
    for k in range(3 * nfull, nchunks):
        m = k % 3
        if k >= 2:
            drain_scatter(k - 2, (k + 1) % 3)
        if k + 1 < nchunks:
            fire_gather(k + 1, (k + 1) % 3)
        drain_gather(k, m)
        fire_scatter(k, m)
    drain_scatter(nchunks - 2, (nchunks - 2) % 3)
    drain_scatter(nchunks - 1, (nchunks - 1) % 3)
    plsc.subcore_barrier()

    def wb_blk(k, _):
        b = s + k * NS

        @pl.when(b < nblk)
        def _():
            pltpu.sync_copy(agg_sh.at[pl.ds(b * ROWBLK, ROWBLK)],
                            out_hbm.at[c, pl.ds(b * ROWBLK, ROWBLK)])
        return 0
    lax.fori_loop(0, blk_iters, wb_blk, 0)


def _segment_sum_sc(h, src, dst):
    n_nodes, feat = h.shape
    e = src.shape[0]
    assert e % NW == 0
    edges_per_w = e // NW
    nchunks = edges_per_w // CHUNK
    assert edges_per_w % CHUNK == 0 and nchunks >= 5
    mesh = plsc.VectorSubcoreMesh(core_axis_name="c", subcore_axis_name="s")
    body = functools.partial(_seg_sum_body, n_nodes, feat, edges_per_w)
    return pl.kernel(
        body,
        out_type=jax.ShapeDtypeStruct((NC, n_nodes, feat), jnp.float32),
        mesh=mesh,
        scratch_types=[
            pltpu.VMEM_SHARED((n_nodes, feat), jnp.float32),
            pltpu.VMEM((edges_per_w,), jnp.int32),
            pltpu.VMEM((edges_per_w,), jnp.int32),
            pltpu.VMEM((3, CHUNK, feat), jnp.float32),
            pltpu.SemaphoreType.DMA,
            pltpu.SemaphoreType.DMA,
            pltpu.SemaphoreType.DMA,
            pltpu.SemaphoreType.DMA,
            pltpu.SemaphoreType.DMA,
            pltpu.SemaphoreType.DMA,
        ],
    )(h, src, dst)


def _mlp_body(h_ref, a0_ref, a1_ref, w_ref, b_ref, o_ref):
    acc = h_ref[...] + a0_ref[...] + a1_ref[...]
    y = jnp.dot(acc, w_ref[...], preferred_element_type=jnp.float32)
    o_ref[...] = jnp.tanh(y + b_ref[...])


def _mlp_final_body(h_ref, a0_ref, a1_ref, w_ref, b_ref, wo_ref, bo_ref, o_ref):
    acc = h_ref[...] + a0_ref[...] + a1_ref[...]
    y = jnp.dot(acc, w_ref[...], preferred_element_type=jnp.float32)
    t = jnp.tanh(y + b_ref[...])
    o_ref[...] = jnp.dot(t, wo_ref[...], preferred_element_type=jnp.float32) + bo_ref[...]


def _mlp_tc(h, a, w, b, wout=None, bout=None):
    n_nodes, feat = h.shape
    blk = 400
    assert n_nodes % blk == 0
    grid = (n_nodes // blk,)
    row_spec = pl.BlockSpec((blk, feat), lambda i: (i, 0))
    full = lambda shape: pl.BlockSpec(shape, lambda i: (0,) * len(shape))
    args = [h, a[0], a[1], w, b.reshape(1, -1)]
    in_specs = [row_spec, row_spec, row_spec, full(w.shape), full((1, feat))]
    if wout is None:
        body, out_cols = _mlp_body, w.shape[1]
    else:
        body, out_cols = _mlp_final_body, wout.shape[1]
        args += [wout, bout.reshape(1, -1)]
        in_specs += [full(wout.shape), full((1, wout.shape[1]))]
    return pl.pallas_call(
        body,
        grid=grid,
        in_specs=in_specs,
        out_specs=pl.BlockSpec((blk, out_cols), lambda i: (i, 0)),
        out_shape=jax.ShapeDtypeStruct((n_nodes, out_cols), jnp.float32),
    )(*args)


def kernel(x, edge_index, W0, b0, W1, b1, W2, b2, Wout, bout):
    src = edge_index[0]
    dst = edge_index[1]
    h = x
    a = _segment_sum_sc(h, src, dst)
    h = _mlp_tc(h, a, W0, b0)
    a = _segment_sum_sc(h, src, dst)
    h = _mlp_tc(h, a, W1, b1)
    a = _segment_sum_sc(h, src, dst)
    return _mlp_tc(h, a, W2, b2, Wout, bout)


# speedup vs baseline: 1.1465x; 1.1465x over previous
import functools

import jax
import jax.numpy as jnp
from jax import lax
from jax.experimental import pallas as pl
from jax.experimental.pallas import tpu as pltpu
from jax.experimental.pallas import tpu_sc as plsc

NC = 2
NS = 16
NW = NC * NS
CHUNK = 80


ROWBLK = 80
GRP = 1


def _seg_sum_body(n_nodes, feat, edges_per_w, n_edges, h_hbm, ei_hbm,
                  out_hbm, agg_sh, flat_v, didx_v, rows_v,
                  gsem_a, gsem_b, gsem_c, ssem_a, ssem_b, ssem_c):
    c = lax.axis_index("c")
    s = lax.axis_index("s")
    wid = c * NS + s
    nchunks = edges_per_w // CHUNK
    base = wid * edges_per_w

    nblk = n_nodes // ROWBLK
    blk_iters = (nblk + NS - 1) // NS

    zsrc = rows_v.at[0]
    vdt = rows_v.dtype
    lanes = 32 if vdt == jnp.bfloat16 else 16

    zero_v = jnp.zeros((lanes,), vdt)
    for r in range(ROWBLK):
        for j in range(feat // lanes):
            zsrc[r, pl.ds(j * lanes, lanes)] = zero_v

    def zero_blk(k, _):
        b = s + k * NS

        @pl.when(b < nblk)
        def _():
            pltpu.sync_copy(zsrc, agg_sh.at[pl.ds(b * ROWBLK, ROWBLK)])
        return 0
    lax.fori_loop(0, blk_iters, zero_blk, 0)

    pltpu.sync_copy(ei_hbm.at[pl.ds(n_edges + base, edges_per_w)], didx_v)
    pltpu.sync_copy(ei_hbm.at[pl.ds(base, edges_per_w)], flat_v)

    plsc.subcore_barrier()

    gsems = (gsem_a, gsem_b, gsem_c)
    ssems = (ssem_a, ssem_b, ssem_c)

    def fire_gather(k, m):
        pltpu.async_copy(
            h_hbm.at[flat_v.at[pl.ds(k * CHUNK, CHUNK)]], rows_v.at[m], gsems[m])

    def drain_gather(k, m):
        pltpu.make_async_copy(
            h_hbm.at[flat_v.at[pl.ds(k * CHUNK, CHUNK)]], rows_v.at[m],
            gsems[m]).wait()

    def fire_scatter(k, m):
        pltpu.async_copy(rows_v.at[m],
                         agg_sh.at[didx_v.at[pl.ds(k * CHUNK, CHUNK)]],
                         ssems[m], add=True)

    def drain_scatter(k, m):
        pltpu.make_async_copy(rows_v.at[m],
                              agg_sh.at[didx_v.at[pl.ds(k * CHUNK, CHUNK)]],
                              ssems[m]).wait()

    fire_gather(0, 0)

    def pipe(i, _):
        for m in range(3):
            k = 3 * i + m
            mn = (m + 1) % 3

            @pl.when(k >= 2)
            def _():
                drain_scatter(k - 2, mn)
            fire_gather(k + 1, mn)
            drain_gather(k, m)
            fire_scatter(k, m)
        return 0
    nfull = (nchunks - 1) // 3
    lax.fori_loop(0, nfull, pipe, 0)
    for k in range(3 * nfull, nchunks):
        m = k % 3
        if k >= 2:
            drain_scatter(k - 2, (k + 1) % 3)
        if k + 1 < nchunks:
            fire_gather(k + 1, (k + 1) % 3)
        drain_gather(k, m)
        fire_scatter(k, m)
    drain_scatter(nchunks - 2, (nchunks - 2) % 3)
    drain_scatter(nchunks - 1, (nchunks - 1) % 3)
    plsc.subcore_barrier()

    def wb_blk(k, _):
        b = s + k * NS

        @pl.when(b < nblk)
        def _():
            pltpu.sync_copy(agg_sh.at[pl.ds(b * ROWBLK, ROWBLK)],
                            out_hbm.at[c, pl.ds(b * ROWBLK, ROWBLK)])
        return 0
    lax.fori_loop(0, blk_iters, wb_blk, 0)


def _segment_sum_sc(h, ei_flat):
    n_nodes, feat = h.shape
    e = ei_flat.shape[0] // 2
    assert e % NW == 0
    edges_per_w = e // NW
    nchunks = edges_per_w // CHUNK
    assert edges_per_w % CHUNK == 0 and nchunks >= 5
    mesh = plsc.VectorSubcoreMesh(core_axis_name="c", subcore_axis_name="s")
    body = functools.partial(_seg_sum_body, n_nodes, feat, edges_per_w, e)
    return pl.kernel(
        body,
        out_type=jax.ShapeDtypeStruct((NC, n_nodes, feat), h.dtype),
        mesh=mesh,
        scratch_types=[
            pltpu.VMEM_SHARED((n_nodes, feat), h.dtype),
            pltpu.VMEM((edges_per_w,), jnp.int32),
            pltpu.VMEM((edges_per_w,), jnp.int32),
            pltpu.VMEM((3, CHUNK, feat), h.dtype),
            pltpu.SemaphoreType.DMA,
            pltpu.SemaphoreType.DMA,
            pltpu.SemaphoreType.DMA,
            pltpu.SemaphoreType.DMA,
            pltpu.SemaphoreType.DMA,
            pltpu.SemaphoreType.DMA,
        ],
    )(h, ei_flat)


def _mlp_body(h_ref, a0_ref, a1_ref, w_ref, b_ref, o_ref):
    acc = h_ref[...] + a0_ref[0] + a1_ref[0]
    y = jnp.dot(acc, w_ref[...], preferred_element_type=jnp.float32)
    o_ref[...] = jnp.tanh(y + b_ref[...])


def _mlp_final_body(h_ref, a0_ref, a1_ref, w_ref, b_ref, wo_ref, bo_ref, o_ref):
    acc = h_ref[...] + a0_ref[0] + a1_ref[0]
    y = jnp.dot(acc, w_ref[...], preferred_element_type=jnp.float32)
    t = jnp.tanh(y + b_ref[...])
    o_ref[...] = jnp.dot(t, wo_ref[...], preferred_element_type=jnp.float32) + bo_ref[...]


def _mlp_tc(h, a, w, b, wout=None, bout=None):
    n_nodes, feat = h.shape
    blk = 1000
    assert n_nodes % blk == 0
    grid = (n_nodes // blk,)
    row_spec = pl.BlockSpec((blk, feat), lambda i: (i, 0))
    a0_spec = pl.BlockSpec((1, blk, feat), lambda i: (0, i, 0))
    a1_spec = pl.BlockSpec((1, blk, feat), lambda i: (1, i, 0))
    full = lambda shape: pl.BlockSpec(shape, lambda i: (0,) * len(shape))
    args = [h, a, a, w, b.reshape(1, -1)]
    in_specs = [row_spec, a0_spec, a1_spec, full(w.shape), full((1, feat))]
    if wout is None:
        body, out_cols = _mlp_body, w.shape[1]
    else:
        body, out_cols = _mlp_final_body, wout.shape[1]
        args += [wout, bout.reshape(1, -1)]
        in_specs += [full(wout.shape), full((1, wout.shape[1]))]
    return pl.pallas_call(
        body,
        grid=grid,
        in_specs=in_specs,
        out_specs=pl.BlockSpec((blk, out_cols), lambda i: (i, 0)),
        out_shape=jax.ShapeDtypeStruct((n_nodes, out_cols), jnp.float32),
    )(*args)


def kernel(x, edge_index, W0, b0, W1, b1, W2, b2, Wout, bout):
    ei_flat = jnp.reshape(edge_index, (-1,))
    h = x
    a = _segment_sum_sc(h, ei_flat)
    h = _mlp_tc(h, a, W0, b0)
    a = _segment_sum_sc(h, ei_flat)
    h = _mlp_tc(h, a, W1, b1)
    a = _segment_sum_sc(h, ei_flat)
    return _mlp_tc(h, a, W2, b2, Wout, bout)


# speedup vs baseline: 1.1672x; 1.0180x over previous
import functools

import jax
import jax.numpy as jnp
from jax import lax
from jax.experimental import pallas as pl
from jax.experimental.pallas import tpu as pltpu
from jax.experimental.pallas import tpu_sc as plsc

NC = 2
NS = 16
NW = NC * NS
CHUNK = 80


ROWBLK = 80
GRP = 1


def _seg_sum_body(n_nodes, feat, edges_per_w, n_edges, h_hbm, ei_hbm,
                  out_hbm, agg_sh, flat_v, didx_v, rows_v,
                  gsem_a, gsem_b, gsem_c, ssem_a, ssem_b, ssem_c):
    c = lax.axis_index("c")
    s = lax.axis_index("s")
    wid = c * NS + s
    nchunks = edges_per_w // CHUNK
    base = wid * edges_per_w

    nblk = n_nodes // ROWBLK
    blk_iters = (nblk + NS - 1) // NS

    zsrc = rows_v.at[0]
    vdt = rows_v.dtype
    lanes = 32 if vdt == jnp.bfloat16 else 16

    zero_v = jnp.zeros((lanes,), vdt)
    for r in range(ROWBLK):
        for j in range(feat // lanes):
            zsrc[r, pl.ds(j * lanes, lanes)] = zero_v

    def zero_blk(k, _):
        b = s + k * NS

        @pl.when(b < nblk)
        def _():
            pltpu.async_copy(zsrc, agg_sh.at[pl.ds(b * ROWBLK, ROWBLK)], gsem_a)
        return 0
    lax.fori_loop(0, blk_iters, zero_blk, 0)

    pltpu.sync_copy(ei_hbm.at[pl.ds(n_edges + base, edges_per_w)], didx_v)
    pltpu.sync_copy(ei_hbm.at[pl.ds(base, edges_per_w)], flat_v)

    def zero_drain(k, _):
        b = s + k * NS

        @pl.when(b < nblk)
        def _():
            pltpu.make_async_copy(
                zsrc, agg_sh.at[pl.ds(b * ROWBLK, ROWBLK)], gsem_a).wait()
        return 0
    lax.fori_loop(0, blk_iters, zero_drain, 0)

    plsc.subcore_barrier()

    gsems = (gsem_a, gsem_b, gsem_c)
    ssems = (ssem_a, ssem_b, ssem_c)

    def fire_gather(k, m):
        pltpu.async_copy(
            h_hbm.at[flat_v.at[pl.ds(k * CHUNK, CHUNK)]], rows_v.at[m], gsems[m])

    def drain_gather(k, m):
        pltpu.make_async_copy(
            h_hbm.at[flat_v.at[pl.ds(k * CHUNK, CHUNK)]], rows_v.at[m],
            gsems[m]).wait()

    def fire_scatter(k, m):
        pltpu.async_copy(rows_v.at[m],
                         agg_sh.at[didx_v.at[pl.ds(k * CHUNK, CHUNK)]],
                         ssems[m], add=True)

    def drain_scatter(k, m):
        pltpu.make_async_copy(rows_v.at[m],
                              agg_sh.at[didx_v.at[pl.ds(k * CHUNK, CHUNK)]],
                              ssems[m]).wait()

    fire_gather(0, 0)

    def pipe(i, _):
        for m in range(3):
            k = 3 * i + m
            mn = (m + 1) % 3

            @pl.when(k >= 2)
            def _():
                drain_scatter(k - 2, mn)
            fire_gather(k + 1, mn)
            drain_gather(k, m)
            fire_scatter(k, m)
        return 0
    nfull = (nchunks - 1) // 3
    lax.fori_loop(0, nfull, pipe, 0)
    for k in range(3 * nfull, nchunks):
        m = k % 3
        if k >= 2:
            drain_scatter(k - 2, (k + 1) % 3)
        if k + 1 < nchunks:
            fire_gather(k + 1, (k + 1) % 3)
        drain_gather(k, m)
        fire_scatter(k, m)
    drain_scatter(nchunks - 2, (nchunks - 2) % 3)
    drain_scatter(nchunks - 1, (nchunks - 1) % 3)
    plsc.subcore_barrier()

    def wb_blk(k, _):
        b = s + k * NS

        @pl.when(b < nblk)
        def _():
            pltpu.async_copy(agg_sh.at[pl.ds(b * ROWBLK, ROWBLK)],
                             out_hbm.at[c, pl.ds(b * ROWBLK, ROWBLK)], gsem_a)
        return 0
    lax.fori_loop(0, blk_iters, wb_blk, 0)

    def wb_drain(k, _):
        b = s + k * NS

        @pl.when(b < nblk)
        def _():
            pltpu.make_async_copy(
                agg_sh.at[pl.ds(b * ROWBLK, ROWBLK)],
                out_hbm.at[c, pl.ds(b * ROWBLK, ROWBLK)], gsem_a).wait()
        return 0
    lax.fori_loop(0, blk_iters, wb_drain, 0)


def _segment_sum_sc(h, ei_flat):
    n_nodes, feat = h.shape
    e = ei_flat.shape[0] // 2
    assert e % NW == 0
    edges_per_w = e // NW
    nchunks = edges_per_w // CHUNK
    assert edges_per_w % CHUNK == 0 and nchunks >= 5
    mesh = plsc.VectorSubcoreMesh(core_axis_name="c", subcore_axis_name="s")
    body = functools.partial(_seg_sum_body, n_nodes, feat, edges_per_w, e)
    return pl.kernel(
        body,
        out_type=jax.ShapeDtypeStruct((NC, n_nodes, feat), h.dtype),
        mesh=mesh,
        scratch_types=[
            pltpu.VMEM_SHARED((n_nodes, feat), h.dtype),
            pltpu.VMEM((edges_per_w,), jnp.int32),
            pltpu.VMEM((edges_per_w,), jnp.int32),
            pltpu.VMEM((3, CHUNK, feat), h.dtype),
            pltpu.SemaphoreType.DMA,
            pltpu.SemaphoreType.DMA,
            pltpu.SemaphoreType.DMA,
            pltpu.SemaphoreType.DMA,
            pltpu.SemaphoreType.DMA,
            pltpu.SemaphoreType.DMA,
        ],
    )(h, ei_flat)


def _mlp_body(h_ref, a0_ref, a1_ref, w_ref, b_ref, o_ref):
    acc = h_ref[...] + a0_ref[0] + a1_ref[0]
    y = jnp.dot(acc, w_ref[...], preferred_element_type=jnp.float32)
    o_ref[...] = jnp.tanh(y + b_ref[...])


def _mlp_final_body(h_ref, a0_ref, a1_ref, w_ref, b_ref, wo_ref, bo_ref, o_ref):
    acc = h_ref[...] + a0_ref[0] + a1_ref[0]
    y = jnp.dot(acc, w_ref[...], preferred_element_type=jnp.float32)
    t = jnp.tanh(y + b_ref[...])
    o_ref[...] = jnp.dot(t, wo_ref[...], preferred_element_type=jnp.float32) + bo_ref[...]


def _mlp_tc(h, a, w, b, wout=None, bout=None):
    n_nodes, feat = h.shape
    blk = 1000
    assert n_nodes % blk == 0
    grid = (n_nodes // blk,)
    row_spec = pl.BlockSpec((blk, feat), lambda i: (i, 0))
    a0_spec = pl.BlockSpec((1, blk, feat), lambda i: (0, i, 0))
    a1_spec = pl.BlockSpec((1, blk, feat), lambda i: (1, i, 0))
    full = lambda shape: pl.BlockSpec(shape, lambda i: (0,) * len(shape))
    args = [h, a, a, w, b.reshape(1, -1)]
    in_specs = [row_spec, a0_spec, a1_spec, full(w.shape), full((1, feat))]
    if wout is None:
        body, out_cols = _mlp_body, w.shape[1]
    else:
        body, out_cols = _mlp_final_body, wout.shape[1]
        args += [wout, bout.reshape(1, -1)]
        in_specs += [full(wout.shape), full((1, wout.shape[1]))]
    return pl.pallas_call(
        body,
        grid=grid,
        in_specs=in_specs,
        out_specs=pl.BlockSpec((blk, out_cols), lambda i: (i, 0)),
        out_shape=jax.ShapeDtypeStruct((n_nodes, out_cols), jnp.float32),
    )(*args)


def kernel(x, edge_index, W0, b0, W1, b1, W2, b2, Wout, bout):
    ei_flat = jnp.reshape(edge_index, (-1,))
    h = x
    a = _segment_sum_sc(h, ei_flat)
    h = _mlp_tc(h, a, W0, b0)
    a = _segment_sum_sc(h, ei_flat)
    h = _mlp_tc(h, a, W1, b1)
    a = _segment_sum_sc(h, ei_flat)
    return _mlp_tc(h, a, W2, b2, Wout, bout)


# speedup vs baseline: 1.2365x; 1.0594x over previous
import functools

import jax
import jax.numpy as jnp
from jax import lax
from jax.experimental import pallas as pl
from jax.experimental.pallas import tpu as pltpu
from jax.experimental.pallas import tpu_sc as plsc

NC = 2
NS = 16
NW = NC * NS
CHUNK = 80


ROWBLK = 80
GRP = 1


def _seg_sum_body(n_nodes, feat, edges_per_w, n_edges, h_hbm, ei_hbm,
                  out_hbm, agg_sh, flat_v, didx_v, rows_v,
                  gsem_a, gsem_b, gsem_c, ssem_a, ssem_b, ssem_c):
    c = lax.axis_index("c")
    s = lax.axis_index("s")
    wid = c * NS + s
    nchunks = edges_per_w // CHUNK
    base = wid * edges_per_w

    nblk = n_nodes // ROWBLK
    blk_iters = (nblk + NS - 1) // NS

    zsrc = rows_v.at[0]
    vdt = rows_v.dtype
    lanes = 32 if vdt == jnp.bfloat16 else 16

    zero_v = jnp.zeros((lanes,), vdt)
    for r in range(ROWBLK):
        for j in range(feat // lanes):
            zsrc[r, pl.ds(j * lanes, lanes)] = zero_v

    def zero_blk(k, _):
        b = s + k * NS

        @pl.when(b < nblk)
        def _():
            pltpu.async_copy(zsrc, agg_sh.at[pl.ds(b * ROWBLK, ROWBLK)], gsem_a)
        return 0
    lax.fori_loop(0, blk_iters, zero_blk, 0)

    pltpu.sync_copy(ei_hbm.at[pl.ds(n_edges + base, edges_per_w)], didx_v)
    pltpu.sync_copy(ei_hbm.at[pl.ds(base, edges_per_w)], flat_v)

    def zero_drain(k, _):
        b = s + k * NS

        @pl.when(b < nblk)
        def _():
            pltpu.make_async_copy(
                zsrc, agg_sh.at[pl.ds(b * ROWBLK, ROWBLK)], gsem_a).wait()
        return 0
    lax.fori_loop(0, blk_iters, zero_drain, 0)

    gsems = (gsem_a, gsem_b, gsem_c)
    ssems = (ssem_a, ssem_b, ssem_c)

    def fire_gather(k, m):
        pltpu.async_copy(
            h_hbm.at[flat_v.at[pl.ds(k * CHUNK, CHUNK)]], rows_v.at[m], gsems[m])

    def drain_gather(k, m):
        pltpu.make_async_copy(
            h_hbm.at[flat_v.at[pl.ds(k * CHUNK, CHUNK)]], rows_v.at[m],
            gsems[m]).wait()

    def fire_scatter(k, m):
        pltpu.async_copy(rows_v.at[m],
                         agg_sh.at[didx_v.at[pl.ds(k * CHUNK, CHUNK)]],
                         ssems[m], add=True)

    def drain_scatter(k, m):
        pltpu.make_async_copy(rows_v.at[m],
                              agg_sh.at[didx_v.at[pl.ds(k * CHUNK, CHUNK)]],
                              ssems[m]).wait()

    fire_gather(0, 0)
    fire_gather(1, 1)
    plsc.subcore_barrier()

    def pipe(i, _):
        for m in range(3):
            k = 3 * i + m
            mn = (m + 2) % 3

            @pl.when(k >= 1)
            def _():
                drain_scatter(k - 1, mn)

            @pl.when(k + 2 < nchunks)
            def _():
                fire_gather(k + 2, mn)
            drain_gather(k, m)
            fire_scatter(k, m)
        return 0
    nfull = (nchunks - 2) // 3
    lax.fori_loop(0, nfull, pipe, 0)
    for k in range(3 * nfull, nchunks):
        drain_scatter(k - 1, (k - 1) % 3)
        drain_gather(k, k % 3)
        fire_scatter(k, k % 3)
    drain_scatter(nchunks - 1, (nchunks - 1) % 3)
    plsc.subcore_barrier()

    def wb_blk(k, _):
        b = s + k * NS

        @pl.when(b < nblk)
        def _():
            pltpu.async_copy(agg_sh.at[pl.ds(b * ROWBLK, ROWBLK)],
                             out_hbm.at[c, pl.ds(b * ROWBLK, ROWBLK)], gsem_a)
        return 0
    lax.fori_loop(0, blk_iters, wb_blk, 0)

    def wb_drain(k, _):
        b = s + k * NS

        @pl.when(b < nblk)
        def _():
            pltpu.make_async_copy(
                agg_sh.at[pl.ds(b * ROWBLK, ROWBLK)],
                out_hbm.at[c, pl.ds(b * ROWBLK, ROWBLK)], gsem_a).wait()
        return 0
    lax.fori_loop(0, blk_iters, wb_drain, 0)


def _segment_sum_sc(h, ei_flat):
    n_nodes, feat = h.shape
    e = ei_flat.shape[0] // 2
    assert e % NW == 0
    edges_per_w = e // NW
    nchunks = edges_per_w // CHUNK
    assert edges_per_w % CHUNK == 0 and nchunks >= 5
    mesh = plsc.VectorSubcoreMesh(core_axis_name="c", subcore_axis_name="s")
    body = functools.partial(_seg_sum_body, n_nodes, feat, edges_per_w, e)
    return pl.kernel(
        body,
        out_type=jax.ShapeDtypeStruct((NC, n_nodes, feat), h.dtype),
        mesh=mesh,
        scratch_types=[
            pltpu.VMEM_SHARED((n_nodes, feat), h.dtype),
            pltpu.VMEM((edges_per_w,), jnp.int32),
            pltpu.VMEM((edges_per_w,), jnp.int32),
            pltpu.VMEM((3, CHUNK, feat), h.dtype),
            pltpu.SemaphoreType.DMA,
            pltpu.SemaphoreType.DMA,
            pltpu.SemaphoreType.DMA,
            pltpu.SemaphoreType.DMA,
            pltpu.SemaphoreType.DMA,
            pltpu.SemaphoreType.DMA,
        ],
    )(h, ei_flat)


def _mlp_body(h_ref, a0_ref, a1_ref, w_ref, b_ref, o_ref):
    acc = h_ref[...] + a0_ref[0] + a1_ref[0]
    y = jnp.dot(acc, w_ref[...], preferred_element_type=jnp.float32)
    o_ref[...] = jnp.tanh(y + b_ref[...])


def _mlp_final_body(h_ref, a0_ref, a1_ref, w_ref, b_ref, wo_ref, bo_ref, o_ref):
    acc = h_ref[...] + a0_ref[0] + a1_ref[0]
    y = jnp.dot(acc, w_ref[...], preferred_element_type=jnp.float32)
    t = jnp.tanh(y + b_ref[...])
    o_ref[...] = jnp.dot(t, wo_ref[...], preferred_element_type=jnp.float32) + bo_ref[...]


def _mlp_tc(h, a, w, b, wout=None, bout=None):
    n_nodes, feat = h.shape
    blk = 2000
    assert n_nodes % blk == 0
    grid = (n_nodes // blk,)
    row_spec = pl.BlockSpec((blk, feat), lambda i: (i, 0))
    a0_spec = pl.BlockSpec((1, blk, feat), lambda i: (0, i, 0))
    a1_spec = pl.BlockSpec((1, blk, feat), lambda i: (1, i, 0))
    full = lambda shape: pl.BlockSpec(shape, lambda i: (0,) * len(shape))
    args = [h, a, a, w, b.reshape(1, -1)]
    in_specs = [row_spec, a0_spec, a1_spec, full(w.shape), full((1, feat))]
    if wout is None:
        body, out_cols = _mlp_body, w.shape[1]
    else:
        body, out_cols = _mlp_final_body, wout.shape[1]
        args += [wout, bout.reshape(1, -1)]
        in_specs += [full(wout.shape), full((1, wout.shape[1]))]
    return pl.pallas_call(
        body,
        grid=grid,
        in_specs=in_specs,
        out_specs=pl.BlockSpec((blk, out_cols), lambda i: (i, 0)),
        out_shape=jax.ShapeDtypeStruct((n_nodes, out_cols), jnp.float32),
    )(*args)


def kernel(x, edge_index, W0, b0, W1, b1, W2, b2, Wout, bout):
    ei_flat = jnp.reshape(edge_index, (-1,))
    h = x
    a = _segment_sum_sc(h, ei_flat)
    h = _mlp_tc(h, a, W0, b0)
    a = _segment_sum_sc(h, ei_flat)
    h = _mlp_tc(h, a, W1, b1)
    a = _segment_sum_sc(h, ei_flat)
    return _mlp_tc(h, a, W2, b2, Wout, bout)


# speedup vs baseline: 1.2367x; 1.0002x over previous
import functools

import jax
import jax.numpy as jnp
from jax import lax
from jax.experimental import pallas as pl
from jax.experimental.pallas import tpu as pltpu
from jax.experimental.pallas import tpu_sc as plsc

NC = 2
NS = 16
NW = NC * NS
CHUNK = 40
NSLOT = 6


ROWBLK = 40
GRP = 1


def _seg_sum_body(n_nodes, feat, edges_per_w, n_edges, h_hbm, ei_hbm,
                  out_hbm, agg_sh, flat_v, didx_v, rows_v, gsem, ssem):
    c = lax.axis_index("c")
    s = lax.axis_index("s")
    wid = c * NS + s
    nchunks = edges_per_w // CHUNK
    base = wid * edges_per_w

    nblk = n_nodes // ROWBLK
    blk_iters = (nblk + NS - 1) // NS

    zsrc = rows_v.at[0]
    vdt = rows_v.dtype
    lanes = 32 if vdt == jnp.bfloat16 else 16

    zero_v = jnp.zeros((lanes,), vdt)
    for r in range(ROWBLK):
        for j in range(feat // lanes):
            zsrc[r, pl.ds(j * lanes, lanes)] = zero_v

    def zero_blk(k, _):
        b = s + k * NS

        @pl.when(b < nblk)
        def _():
            pltpu.async_copy(zsrc, agg_sh.at[pl.ds(b * ROWBLK, ROWBLK)],
                             gsem.at[0])
        return 0
    lax.fori_loop(0, blk_iters, zero_blk, 0)

    pltpu.sync_copy(ei_hbm.at[pl.ds(n_edges + base, edges_per_w)], didx_v)
    pltpu.sync_copy(ei_hbm.at[pl.ds(base, edges_per_w)], flat_v)

    def zero_drain(k, _):
        b = s + k * NS

        @pl.when(b < nblk)
        def _():
            pltpu.make_async_copy(
                zsrc, agg_sh.at[pl.ds(b * ROWBLK, ROWBLK)], gsem.at[0]).wait()
        return 0
    lax.fori_loop(0, blk_iters, zero_drain, 0)

    def fire_gather(k, m):
        pltpu.async_copy(
            h_hbm.at[flat_v.at[pl.ds(k * CHUNK, CHUNK)]], rows_v.at[m],
            gsem.at[m])

    def drain_gather(k, m):
        pltpu.make_async_copy(
            h_hbm.at[flat_v.at[pl.ds(k * CHUNK, CHUNK)]], rows_v.at[m],
            gsem.at[m]).wait()

    def fire_scatter(k, m):
        pltpu.async_copy(rows_v.at[m],
                         agg_sh.at[didx_v.at[pl.ds(k * CHUNK, CHUNK)]],
                         ssem.at[m], add=True)

    def drain_scatter(k, m):
        pltpu.make_async_copy(rows_v.at[m],
                              agg_sh.at[didx_v.at[pl.ds(k * CHUNK, CHUNK)]],
                              ssem.at[m]).wait()

    PRE = NSLOT // 2
    for k in range(PRE):
        fire_gather(k, k)
    plsc.subcore_barrier()

    def pipe(i, _):
        for m in range(NSLOT):
            k = NSLOT * i + m
            mn = (m + PRE) % NSLOT

            @pl.when(k >= NSLOT - PRE)
            def _():
                drain_scatter(k - (NSLOT - PRE), mn)

            @pl.when(k + PRE < nchunks)
            def _():
                fire_gather(k + PRE, mn)
            drain_gather(k, m)
            fire_scatter(k, m)
        return 0
    nfull = (nchunks - PRE) // NSLOT
    lax.fori_loop(0, nfull, pipe, 0)
    for k in range(NSLOT * nfull, nchunks):
        drain_scatter(k - (NSLOT - PRE), (k - (NSLOT - PRE)) % NSLOT)
        if k + PRE < nchunks:
            fire_gather(k + PRE, (k + PRE) % NSLOT)
        drain_gather(k, k % NSLOT)
        fire_scatter(k, k % NSLOT)
    for k in range(nchunks - (NSLOT - PRE), nchunks):
        drain_scatter(k, k % NSLOT)
    plsc.subcore_barrier()

    def wb_blk(k, _):
        b = s + k * NS

        @pl.when(b < nblk)
        def _():
            pltpu.async_copy(agg_sh.at[pl.ds(b * ROWBLK, ROWBLK)],
                             out_hbm.at[c, pl.ds(b * ROWBLK, ROWBLK)], gsem.at[0])
        return 0
    lax.fori_loop(0, blk_iters, wb_blk, 0)

    def wb_drain(k, _):
        b = s + k * NS

        @pl.when(b < nblk)
        def _():
            pltpu.make_async_copy(
                agg_sh.at[pl.ds(b * ROWBLK, ROWBLK)],
                out_hbm.at[c, pl.ds(b * ROWBLK, ROWBLK)], gsem.at[0]).wait()
        return 0
    lax.fori_loop(0, blk_iters, wb_drain, 0)


def _segment_sum_sc(h, ei_flat):
    n_nodes, feat = h.shape
    e = ei_flat.shape[0] // 2
    assert e % NW == 0
    edges_per_w = e // NW
    nchunks = edges_per_w // CHUNK
    assert edges_per_w % CHUNK == 0 and nchunks >= 5
    mesh = plsc.VectorSubcoreMesh(core_axis_name="c", subcore_axis_name="s")
    body = functools.partial(_seg_sum_body, n_nodes, feat, edges_per_w, e)
    return pl.kernel(
        body,
        out_type=jax.ShapeDtypeStruct((NC, n_nodes, feat), h.dtype),
        mesh=mesh,
        scratch_types=[
            pltpu.VMEM_SHARED((n_nodes, feat), h.dtype),
            pltpu.VMEM((edges_per_w,), jnp.int32),
            pltpu.VMEM((edges_per_w,), jnp.int32),
            pltpu.VMEM((NSLOT, CHUNK, feat), h.dtype),
            pltpu.SemaphoreType.DMA((NSLOT,)),
            pltpu.SemaphoreType.DMA((NSLOT,)),
        ],
    )(h, ei_flat)


def _mlp_body(h_ref, a0_ref, a1_ref, w_ref, b_ref, o_ref):
    acc = h_ref[...] + a0_ref[0] + a1_ref[0]
    y = jnp.dot(acc, w_ref[...], preferred_element_type=jnp.float32)
    o_ref[...] = jnp.tanh(y + b_ref[...])


def _mlp_final_body(h_ref, a0_ref, a1_ref, w_ref, b_ref, wo_ref, bo_ref, o_ref):
    acc = h_ref[...] + a0_ref[0] + a1_ref[0]
    y = jnp.dot(acc, w_ref[...], preferred_element_type=jnp.float32)
    t = jnp.tanh(y + b_ref[...])
    o_ref[...] = jnp.dot(t, wo_ref[...], preferred_element_type=jnp.float32) + bo_ref[...]


def _mlp_tc(h, a, w, b, wout=None, bout=None):
    n_nodes, feat = h.shape
    blk = 2000
    assert n_nodes % blk == 0
    grid = (n_nodes // blk,)
    row_spec = pl.BlockSpec((blk, feat), lambda i: (i, 0))
    a0_spec = pl.BlockSpec((1, blk, feat), lambda i: (0, i, 0))
    a1_spec = pl.BlockSpec((1, blk, feat), lambda i: (1, i, 0))
    full = lambda shape: pl.BlockSpec(shape, lambda i: (0,) * len(shape))
    args = [h, a, a, w, b.reshape(1, -1)]
    in_specs = [row_spec, a0_spec, a1_spec, full(w.shape), full((1, feat))]
    if wout is None:
        body, out_cols = _mlp_body, w.shape[1]
    else:
        body, out_cols = _mlp_final_body, wout.shape[1]
        args += [wout, bout.reshape(1, -1)]
        in_specs += [full(wout.shape), full((1, wout.shape[1]))]
    return pl.pallas_call(
        body,
        grid=grid,
        in_specs=in_specs,
        out_specs=pl.BlockSpec((blk, out_cols), lambda i: (i, 0)),
        out_shape=jax.ShapeDtypeStruct((n_nodes, out_cols), jnp.float32),
    )(*args)


def kernel(x, edge_index, W0, b0, W1, b1, W2, b2, Wout, bout):
    ei_flat = jnp.reshape(edge_index, (-1,))
    h = x
    a = _segment_sum_sc(h, ei_flat)
    h = _mlp_tc(h, a, W0, b0)
    a = _segment_sum_sc(h, ei_flat)
    h = _mlp_tc(h, a, W1, b1)
    a = _segment_sum_sc(h, ei_flat)
    return _mlp_tc(h, a, W2, b2, Wout, bout)


# speedup vs baseline: 1.2370x; 1.0003x over previous
import functools

import jax
import jax.numpy as jnp
from jax import lax
from jax.experimental import pallas as pl
from jax.experimental.pallas import tpu as pltpu
from jax.experimental.pallas import tpu_sc as plsc

NC = 2
NS = 16
NW = NC * NS
CHUNK = 40
NSLOT = 6


ROWBLK = 40


def _seg_sum_body(n_nodes, feat, edges_per_w, n_edges, h_hbm, ei_hbm,
                  out_hbm, agg_sh, flat_v, didx_v, rows_v, gsem, ssem):
    c = lax.axis_index("c")
    s = lax.axis_index("s")
    wid = c * NS + s
    nchunks = edges_per_w // CHUNK
    base = wid * edges_per_w

    nblk = n_nodes // ROWBLK
    blk_iters = (nblk + NS - 1) // NS

    zsrc = rows_v.at[0]
    vdt = rows_v.dtype
    lanes = 32 if vdt == jnp.bfloat16 else 16

    zero_v = jnp.zeros((lanes,), vdt)
    for r in range(ROWBLK):
        for j in range(feat // lanes):
            zsrc[r, pl.ds(j * lanes, lanes)] = zero_v

    def zero_blk(k, _):
        b = s + k * NS

        @pl.when(b < nblk)
        def _():
            pltpu.async_copy(zsrc, agg_sh.at[pl.ds(b * ROWBLK, ROWBLK)],
                             gsem.at[0])
        return 0
    lax.fori_loop(0, blk_iters, zero_blk, 0)

    pltpu.sync_copy(ei_hbm.at[pl.ds(n_edges + base, edges_per_w)], didx_v)
    pltpu.sync_copy(ei_hbm.at[pl.ds(base, edges_per_w)], flat_v)

    def zero_drain(k, _):
        b = s + k * NS

        @pl.when(b < nblk)
        def _():
            pltpu.make_async_copy(
                zsrc, agg_sh.at[pl.ds(b * ROWBLK, ROWBLK)], gsem.at[0]).wait()
        return 0
    lax.fori_loop(0, blk_iters, zero_drain, 0)

    def fire_gather(k, m):
        pltpu.async_copy(
            h_hbm.at[flat_v.at[pl.ds(k * CHUNK, CHUNK)]], rows_v.at[m],
            gsem.at[m])

    def drain_gather(k, m):
        pltpu.make_async_copy(
            h_hbm.at[flat_v.at[pl.ds(k * CHUNK, CHUNK)]], rows_v.at[m],
            gsem.at[m]).wait()

    def fire_scatter(k, m):
        pltpu.async_copy(rows_v.at[m],
                         agg_sh.at[didx_v.at[pl.ds(k * CHUNK, CHUNK)]],
                         ssem.at[m], add=True)

    def drain_scatter(k, m):
        pltpu.make_async_copy(rows_v.at[m],
                              agg_sh.at[didx_v.at[pl.ds(k * CHUNK, CHUNK)]],
                              ssem.at[m]).wait()

    PRE = NSLOT // 2
    for k in range(PRE):
        fire_gather(k, k)
    plsc.subcore_barrier()

    def pipe(i, _):
        for m in range(NSLOT):
            k = NSLOT * i + m
            mn = (m + PRE) % NSLOT

            @pl.when(k >= NSLOT - PRE)
            def _():
                drain_scatter(k - (NSLOT - PRE), mn)

            @pl.when(k + PRE < nchunks)
            def _():
                fire_gather(k + PRE, mn)
            drain_gather(k, m)
            fire_scatter(k, m)
        return 0
    nfull = (nchunks - PRE) // NSLOT
    lax.fori_loop(0, nfull, pipe, 0)
    for k in range(NSLOT * nfull, nchunks):
        drain_scatter(k - (NSLOT - PRE), (k - (NSLOT - PRE)) % NSLOT)
        if k + PRE < nchunks:
            fire_gather(k + PRE, (k + PRE) % NSLOT)
        drain_gather(k, k % NSLOT)
        fire_scatter(k, k % NSLOT)
    for k in range(nchunks - (NSLOT - PRE), nchunks):
        drain_scatter(k, k % NSLOT)
    plsc.subcore_barrier()

    def wb_blk(k, _):
        b = s + k * NS

        @pl.when(b < nblk)
        def _():
            pltpu.async_copy(agg_sh.at[pl.ds(b * ROWBLK, ROWBLK)],
                             out_hbm.at[c, pl.ds(b * ROWBLK, ROWBLK)], gsem.at[0])
        return 0
    lax.fori_loop(0, blk_iters, wb_blk, 0)

    def wb_drain(k, _):
        b = s + k * NS

        @pl.when(b < nblk)
        def _():
            pltpu.make_async_copy(
                agg_sh.at[pl.ds(b * ROWBLK, ROWBLK)],
                out_hbm.at[c, pl.ds(b * ROWBLK, ROWBLK)], gsem.at[0]).wait()
        return 0
    lax.fori_loop(0, blk_iters, wb_drain, 0)


def _segment_sum_sc(h, ei_flat):
    n_nodes, feat = h.shape
    e = ei_flat.shape[0] // 2
    assert e % NW == 0
    edges_per_w = e // NW
    nchunks = edges_per_w // CHUNK
    assert edges_per_w % CHUNK == 0 and nchunks >= 5
    mesh = plsc.VectorSubcoreMesh(core_axis_name="c", subcore_axis_name="s")
    body = functools.partial(_seg_sum_body, n_nodes, feat, edges_per_w, e)
    return pl.kernel(
        body,
        out_type=jax.ShapeDtypeStruct((NC, n_nodes, feat), h.dtype),
        mesh=mesh,
        scratch_types=[
            pltpu.VMEM_SHARED((n_nodes, feat), h.dtype),
            pltpu.VMEM((edges_per_w,), jnp.int32),
            pltpu.VMEM((edges_per_w,), jnp.int32),
            pltpu.VMEM((NSLOT, CHUNK, feat), h.dtype),
            pltpu.SemaphoreType.DMA((NSLOT,)),
            pltpu.SemaphoreType.DMA((NSLOT,)),
        ],
    )(h, ei_flat)


def _mlp_body(h_ref, a0_ref, a1_ref, w_ref, b_ref, o_ref):
    acc = h_ref[...] + a0_ref[0] + a1_ref[0]
    y = jnp.dot(acc, w_ref[...], preferred_element_type=jnp.float32)
    o_ref[...] = jnp.tanh(y + b_ref[...])


def _mlp_final_body(h_ref, a0_ref, a1_ref, w_ref, b_ref, wo_ref, bo_ref, o_ref):
    acc = h_ref[...] + a0_ref[0] + a1_ref[0]
    y = jnp.dot(acc, w_ref[...], preferred_element_type=jnp.float32)
    t = jnp.tanh(y + b_ref[...])
    o_ref[...] = jnp.dot(t, wo_ref[...], preferred_element_type=jnp.float32) + bo_ref[...]


def _mlp_tc(h, a, w, b, wout=None, bout=None):
    n_nodes, feat = h.shape
    blk = 2000
    assert n_nodes % blk == 0
    grid = (n_nodes // blk,)
    row_spec = pl.BlockSpec((blk, feat), lambda i: (i, 0))
    a0_spec = pl.BlockSpec((1, blk, feat), lambda i: (0, i, 0))
    a1_spec = pl.BlockSpec((1, blk, feat), lambda i: (1, i, 0))
    full = lambda shape: pl.BlockSpec(shape, lambda i: (0,) * len(shape))
    args = [h, a, a, w, b.reshape(1, -1)]
    in_specs = [row_spec, a0_spec, a1_spec, full(w.shape), full((1, feat))]
    if wout is None:
        body, out_cols = _mlp_body, w.shape[1]
    else:
        body, out_cols = _mlp_final_body, wout.shape[1]
        args += [wout, bout.reshape(1, -1)]
        in_specs += [full(wout.shape), full((1, wout.shape[1]))]
    return pl.pallas_call(
        body,
        grid=grid,
        in_specs=in_specs,
        out_specs=pl.BlockSpec((blk, out_cols), lambda i: (i, 0)),
        out_shape=jax.ShapeDtypeStruct((n_nodes, out_cols), jnp.float32),
    )(*args)


def kernel(x, edge_index, W0, b0, W1, b1, W2, b2, Wout, bout):
    ei_flat = jnp.reshape(edge_index, (-1,))
    h = x
    a = _segment_sum_sc(h, ei_flat)
    h = _mlp_tc(h, a, W0, b0)
    a = _segment_sum_sc(h, ei_flat)
    h = _mlp_tc(h, a, W1, b1)
    a = _segment_sum_sc(h, ei_flat)
    return _mlp_tc(h, a, W2, b2, Wout, bout)
